# Initial kernel scaffold; baseline (speedup 1.0000x reference)
import jax
import jax.numpy as jnp
from jax.experimental import pallas as pl


def kernel(inputs_embeds, pos_table, ln_gamma, ln_beta):
    raise NotImplementedError("write your pallas kernel here")


# speedup vs baseline: 2.1499x; 2.1499x over previous
import jax
import jax.numpy as jnp
from jax.experimental import pallas as pl

_EPS = 1e-12
_BLOCK_S = 512


def _ln_add_kernel(x_ref, pos_ref, gamma_ref, beta_ref, out_ref):
    x = x_ref[...]
    p = pos_ref[...]
    e = x + p[None, :, :]
    mean = jnp.mean(e, axis=-1, keepdims=True)
    c = e - mean
    var = jnp.mean(c * c, axis=-1, keepdims=True)
    inv = jax.lax.rsqrt(var + _EPS)
    out_ref[...] = c * inv * gamma_ref[...][None] + beta_ref[...][None]


def kernel(inputs_embeds, pos_table, ln_gamma, ln_beta):
    B, S, H = inputs_embeds.shape
    bs = _BLOCK_S
    grid = (S // bs, B)
    return pl.pallas_call(
        _ln_add_kernel,
        grid=grid,
        in_specs=[
            pl.BlockSpec((1, bs, H), lambda j, b: (b, j, 0)),
            pl.BlockSpec((bs, H), lambda j, b: (j, 0)),
            pl.BlockSpec((1, H), lambda j, b: (0, 0)),
            pl.BlockSpec((1, H), lambda j, b: (0, 0)),
        ],
        out_specs=pl.BlockSpec((1, bs, H), lambda j, b: (b, j, 0)),
        out_shape=jax.ShapeDtypeStruct((B, S, H), jnp.float32),
    )(inputs_embeds, pos_table, ln_gamma.reshape(1, H), ln_beta.reshape(1, H))


# speedup vs baseline: 2.4499x; 1.1395x over previous
import jax
import jax.numpy as jnp
from jax.experimental import pallas as pl

_EPS = 1e-12
_BLOCK_S = 1024


def _ln_add_kernel(x_ref, pos_ref, gamma_ref, beta_ref, out_ref):
    x = x_ref[...]
    p = pos_ref[...]
    e = x + p[None, :, :]
    mean = jnp.mean(e, axis=-1, keepdims=True)
    c = e - mean
    var = jnp.mean(c * c, axis=-1, keepdims=True)
    inv = jax.lax.rsqrt(var + _EPS)
    out_ref[...] = c * inv * gamma_ref[...][None] + beta_ref[...][None]


def kernel(inputs_embeds, pos_table, ln_gamma, ln_beta):
    B, S, H = inputs_embeds.shape
    bs = _BLOCK_S
    grid = (S // bs, B)
    return pl.pallas_call(
        _ln_add_kernel,
        grid=grid,
        in_specs=[
            pl.BlockSpec((1, bs, H), lambda j, b: (b, j, 0)),
            pl.BlockSpec((bs, H), lambda j, b: (j, 0)),
            pl.BlockSpec((1, H), lambda j, b: (0, 0)),
            pl.BlockSpec((1, H), lambda j, b: (0, 0)),
        ],
        out_specs=pl.BlockSpec((1, bs, H), lambda j, b: (b, j, 0)),
        out_shape=jax.ShapeDtypeStruct((B, S, H), jnp.float32),
    )(inputs_embeds, pos_table, ln_gamma.reshape(1, H), ln_beta.reshape(1, H))


# speedup vs baseline: 2.6794x; 1.0937x over previous
import jax
import jax.numpy as jnp
from jax.experimental import pallas as pl

_EPS = 1e-12
_BLOCK_S = 2048


def _ln_add_kernel(x_ref, pos_ref, gamma_ref, beta_ref, out_ref):
    x = x_ref[...]
    p = pos_ref[...]
    e = x + p[None, :, :]
    mean = jnp.mean(e, axis=-1, keepdims=True)
    c = e - mean
    var = jnp.mean(c * c, axis=-1, keepdims=True)
    inv = jax.lax.rsqrt(var + _EPS)
    out_ref[...] = c * inv * gamma_ref[...][None] + beta_ref[...][None]


def kernel(inputs_embeds, pos_table, ln_gamma, ln_beta):
    B, S, H = inputs_embeds.shape
    bs = _BLOCK_S
    grid = (S // bs, B)
    return pl.pallas_call(
        _ln_add_kernel,
        grid=grid,
        in_specs=[
            pl.BlockSpec((1, bs, H), lambda j, b: (b, j, 0)),
            pl.BlockSpec((bs, H), lambda j, b: (j, 0)),
            pl.BlockSpec((1, H), lambda j, b: (0, 0)),
            pl.BlockSpec((1, H), lambda j, b: (0, 0)),
        ],
        out_specs=pl.BlockSpec((1, bs, H), lambda j, b: (b, j, 0)),
        out_shape=jax.ShapeDtypeStruct((B, S, H), jnp.float32),
    )(inputs_embeds, pos_table, ln_gamma.reshape(1, H), ln_beta.reshape(1, H))
